# baseline (speedup 1.0000x reference)
import functools

import jax
import jax.numpy as jnp
from jax.experimental import pallas as pl
from jax.experimental.pallas import tpu as pltpu
from jax.experimental.pallas import tpu_sc as plsc

N = 4096
F = 768
K = 8
C = 8
KC = K * C
LOG2PI = 1.8378770664093453
R = 2048
G = N // R

_SC_NC = 1
_SC_NS = 8
_SC_NW = _SC_NC * _SC_NS
_SC_CHUNK = N // _SC_NW
_SC_L = 16


def _tc_body(x_ref, cent_ref, mu_ref, lv_ref, lw_ref,
             pack_ref, p_ref, m2_ref, bias_ref, z_ref, cb_ref):
    pid = pl.program_id(0)

    @pl.when(pid == 0)
    def _prep():
        lv = lv_ref[...]
        mu = mu_ref[...]
        p = jnp.exp(-lv)
        m2 = mu * p
        p_ref[...] = (-0.5 * p).astype(jnp.bfloat16)
        m2_ref[0:KC, :] = m2.astype(jnp.bfloat16)
        m2_ref[KC:KC + K, :] = cent_ref[...].astype(jnp.bfloat16)
        bias_ref[...] = (-0.5 * (jnp.sum(mu * m2 + lv, axis=1, keepdims=True)
                                 + F * LOG2PI) + lw_ref[...])
        zs = []
        for k in range(K):
            g = lw_ref[k * C:(k + 1) * C, :]
            m = jnp.max(g, axis=0, keepdims=True)
            zs.append(m + jnp.log(jnp.sum(jnp.exp(g - m), axis=0,
                                          keepdims=True)))
        z_ref[...] = jnp.concatenate(zs, axis=0)
        c = cent_ref[...]
        cb_ref[...] = -0.5 * jnp.sum(c * c, axis=1, keepdims=True)

    x = x_ref[...]
    x_bf = x.astype(jnp.bfloat16)
    xsq_bf = (x * x).astype(jnp.bfloat16)

    dot_bf = functools.partial(
        jax.lax.dot_general,
        dimension_numbers=(((1,), (1,)), ((), ())),
        preferred_element_type=jnp.float32,
    )
    s1t = dot_bf(p_ref[...], xsq_bf)
    s2t = dot_bf(m2_ref[...], x_bf)
    comp = s1t + s2t[0:KC, :] + bias_ref[...]

    pack_ref[K:2 * K, :] = s2t[KC:KC + K, :] + cb_ref[...]

    lses = []
    for k in range(K):
        g = comp[k * C:(k + 1) * C, :]
        m = jnp.max(g, axis=0, keepdims=True)
        lses.append(m + jnp.log(jnp.sum(jnp.exp(g - m), axis=0,
                                        keepdims=True)))
    pack_ref[0:K, :] = jnp.concatenate(lses, axis=0) - z_ref[...]


def _sc_route_body(pack_hbm, out_hbm, pack_v, out_v):
    wid = jax.lax.axis_index("s") * _SC_NC + jax.lax.axis_index("c")
    base = wid * _SC_CHUNK
    cols = pl.ds(base, _SC_CHUNK)
    pltpu.sync_copy(pack_hbm.at[:, cols], pack_v)
    for i in range(_SC_CHUNK // _SC_L):
        s = pl.ds(_SC_L * i, _SC_L)
        best = pack_v[K, s]
        pick = pack_v[0, s]
        for k in range(1, K):
            v = pack_v[K + k, s]
            m = v > best
            pick = jnp.where(m, pack_v[k, s], pick)
            best = jnp.where(m, v, best)
        out_v[s] = pick
    pltpu.sync_copy(out_v, out_hbm.at[cols])


def kernel(x, centroids, means, logvars, logweights):
    mu = means.reshape(KC, F)
    lv = logvars.reshape(KC, F)
    lw = logweights.reshape(KC, 1)
    pack = pl.pallas_call(
        _tc_body,
        grid=(G,),
        in_specs=[
            pl.BlockSpec((R, F), lambda i: (i, 0)),
            pl.BlockSpec((K, F), lambda i: (0, 0)),
            pl.BlockSpec((KC, F), lambda i: (0, 0)),
            pl.BlockSpec((KC, F), lambda i: (0, 0)),
            pl.BlockSpec((KC, 1), lambda i: (0, 0)),
        ],
        out_specs=pl.BlockSpec((2 * K, R), lambda i: (0, i)),
        out_shape=jax.ShapeDtypeStruct((2 * K, N), jnp.float32),
        scratch_shapes=[
            pltpu.VMEM((KC, F), jnp.bfloat16),
            pltpu.VMEM((KC + K, F), jnp.bfloat16),
            pltpu.VMEM((KC, 1), jnp.float32),
            pltpu.VMEM((K, 1), jnp.float32),
            pltpu.VMEM((K, 1), jnp.float32),
        ],
    )(x, centroids, mu, lv, lw)

    mesh = plsc.VectorSubcoreMesh(core_axis_name="c", subcore_axis_name="s",
                                  num_cores=_SC_NC, num_subcores=_SC_NS)
    out = pl.kernel(
        _sc_route_body,
        out_type=jax.ShapeDtypeStruct((N,), jnp.float32),
        mesh=mesh,
        scratch_types=[
            pltpu.VMEM((2 * K, _SC_CHUNK), jnp.float32),
            pltpu.VMEM((_SC_CHUNK,), jnp.float32),
        ],
    )(pack)
    return out


# speedup vs baseline: 1.0501x; 1.0501x over previous
import functools

import jax
import jax.numpy as jnp
from jax.experimental import pallas as pl
from jax.experimental.pallas import tpu as pltpu
from jax.experimental.pallas import tpu_sc as plsc

N = 4096
F = 768
K = 8
C = 8
KC = K * C
LOG2PI = 1.8378770664093453
R = 2048
G = N // R

_SC_NC = 1
_SC_NS = 16
_SC_NW = _SC_NC * _SC_NS
_SC_CHUNK = N // _SC_NW
_SC_L = 16


def _tc_body(x_ref, cent_ref, mu_ref, lv_ref, lw_ref,
             pack_ref, p_ref, m2_ref, bias_ref, z_ref, cb_ref):
    pid = pl.program_id(0)

    @pl.when(pid == 0)
    def _prep():
        lv = lv_ref[...]
        mu = mu_ref[...]
        p = jnp.exp(-lv)
        m2 = mu * p
        p_ref[...] = (-0.5 * p).astype(jnp.bfloat16)
        m2_ref[0:KC, :] = m2.astype(jnp.bfloat16)
        m2_ref[KC:KC + K, :] = cent_ref[...].astype(jnp.bfloat16)
        bias_ref[...] = (-0.5 * (jnp.sum(mu * m2 + lv, axis=1, keepdims=True)
                                 + F * LOG2PI) + lw_ref[...])
        zs = []
        for k in range(K):
            g = lw_ref[k * C:(k + 1) * C, :]
            m = jnp.max(g, axis=0, keepdims=True)
            zs.append(m + jnp.log(jnp.sum(jnp.exp(g - m), axis=0,
                                          keepdims=True)))
        z_ref[...] = jnp.concatenate(zs, axis=0)
        c = cent_ref[...]
        cb_ref[...] = -0.5 * jnp.sum(c * c, axis=1, keepdims=True)

    x = x_ref[...]
    x_bf = x.astype(jnp.bfloat16)
    xsq_bf = (x * x).astype(jnp.bfloat16)

    dot_bf = functools.partial(
        jax.lax.dot_general,
        dimension_numbers=(((1,), (1,)), ((), ())),
        preferred_element_type=jnp.float32,
    )
    s1t = dot_bf(p_ref[...], xsq_bf)
    s2t = dot_bf(m2_ref[...], x_bf)
    comp = s1t + s2t[0:KC, :] + bias_ref[...]

    pack_ref[K:2 * K, :] = s2t[KC:KC + K, :] + cb_ref[...]

    lses = []
    for k in range(K):
        g = comp[k * C:(k + 1) * C, :]
        m = jnp.max(g, axis=0, keepdims=True)
        lses.append(m + jnp.log(jnp.sum(jnp.exp(g - m), axis=0,
                                        keepdims=True)))
    pack_ref[0:K, :] = jnp.concatenate(lses, axis=0) - z_ref[...]


def _sc_route_body(pack_hbm, out_hbm, pack_v, out_v):
    wid = jax.lax.axis_index("s") * _SC_NC + jax.lax.axis_index("c")
    base = wid * _SC_CHUNK
    cols = pl.ds(base, _SC_CHUNK)
    pltpu.sync_copy(pack_hbm.at[:, cols], pack_v)
    for i in range(_SC_CHUNK // _SC_L):
        s = pl.ds(_SC_L * i, _SC_L)
        best = pack_v[K, s]
        pick = pack_v[0, s]
        for k in range(1, K):
            v = pack_v[K + k, s]
            m = v > best
            pick = jnp.where(m, pack_v[k, s], pick)
            best = jnp.where(m, v, best)
        out_v[s] = pick
    pltpu.sync_copy(out_v, out_hbm.at[cols])


def kernel(x, centroids, means, logvars, logweights):
    mu = means.reshape(KC, F)
    lv = logvars.reshape(KC, F)
    lw = logweights.reshape(KC, 1)
    pack = pl.pallas_call(
        _tc_body,
        grid=(G,),
        in_specs=[
            pl.BlockSpec((R, F), lambda i: (i, 0)),
            pl.BlockSpec((K, F), lambda i: (0, 0)),
            pl.BlockSpec((KC, F), lambda i: (0, 0)),
            pl.BlockSpec((KC, F), lambda i: (0, 0)),
            pl.BlockSpec((KC, 1), lambda i: (0, 0)),
        ],
        out_specs=pl.BlockSpec((2 * K, R), lambda i: (0, i)),
        out_shape=jax.ShapeDtypeStruct((2 * K, N), jnp.float32),
        scratch_shapes=[
            pltpu.VMEM((KC, F), jnp.bfloat16),
            pltpu.VMEM((KC + K, F), jnp.bfloat16),
            pltpu.VMEM((KC, 1), jnp.float32),
            pltpu.VMEM((K, 1), jnp.float32),
            pltpu.VMEM((K, 1), jnp.float32),
        ],
    )(x, centroids, mu, lv, lw)

    mesh = plsc.VectorSubcoreMesh(core_axis_name="c", subcore_axis_name="s",
                                  num_cores=_SC_NC)
    out = pl.kernel(
        _sc_route_body,
        out_type=jax.ShapeDtypeStruct((N,), jnp.float32),
        mesh=mesh,
        scratch_types=[
            pltpu.VMEM((2 * K, _SC_CHUNK), jnp.float32),
            pltpu.VMEM((_SC_CHUNK,), jnp.float32),
        ],
    )(pack)
    return out
